# baseline (speedup 1.0000x reference)
import functools

import jax
import jax.numpy as jnp
from jax import lax
from jax.experimental import pallas as pl
from jax.experimental.pallas import tpu as pltpu
from jax.experimental.pallas import tpu_sc as plsc

N = 10000
D = 128
E = 320000
NPAD = 10240
TRASH = 10000
NC = 2
NS = 16
DH = D // NC
CHUNK = 128
CPT = 158
EPT = CPT * CHUNK
EPAD = NS * EPT
RPT = NPAD // NS
SW = 16


def _mm_out(hw, as_ref, ad_ref, hwst_ref, als_ref, ald_ref):
    hwst_ref[0] = hw[:, :DH]
    hwst_ref[1] = hw[:, DH:]
    als_ref[...] = jnp.dot(hw, as_ref[...], preferred_element_type=jnp.float32)
    ald_ref[...] = jnp.dot(hw, ad_ref[...], preferred_element_type=jnp.float32)


def _dense_body(h_ref, w_ref, as_ref, ad_ref, hwst_ref, als_ref, ald_ref):
    hw = jnp.dot(h_ref[...], w_ref[...], preferred_element_type=jnp.float32)
    _mm_out(hw, as_ref, ad_ref, hwst_ref, als_ref, ald_ref)


def _dense(h, W, a_s, a_d, blk=2048):
    grid = (NPAD // blk,)
    return pl.pallas_call(
        _dense_body,
        grid=grid,
        in_specs=[
            pl.BlockSpec((blk, D), lambda i: (i, 0)),
            pl.BlockSpec((D, D), lambda i: (0, 0)),
            pl.BlockSpec((D, 1), lambda i: (0, 0)),
            pl.BlockSpec((D, 1), lambda i: (0, 0)),
        ],
        out_specs=[
            pl.BlockSpec((NC, blk, DH), lambda i: (0, i, 0)),
            pl.BlockSpec((blk, 1), lambda i: (i, 0)),
            pl.BlockSpec((blk, 1), lambda i: (i, 0)),
        ],
        out_shape=[
            jax.ShapeDtypeStruct((NC, NPAD, DH), jnp.float32),
            jax.ShapeDtypeStruct((NPAD, 1), jnp.float32),
            jax.ShapeDtypeStruct((NPAD, 1), jnp.float32),
        ],
    )(h, W, a_s.reshape(D, 1), a_d.reshape(D, 1))


def _h_block(ah_ref, asc_ref, hwst_ref, als_ref, ald_ref, b_ref, p_ref):
    e = als_ref[...] + ald_ref[...]
    wself = jnp.exp(jnp.maximum(e, 0.0) + 0.2 * jnp.minimum(e, 0.0))
    hw = jnp.concatenate([hwst_ref[0], hwst_ref[1]], axis=-1)
    acc = jnp.concatenate([ah_ref[0], ah_ref[1]], axis=-1)
    num = acc + wself * hw
    den = asc_ref[:, 0:1] + wself
    h = num / (den + 1e-16) + b_ref[...]
    p = p_ref[0, 0]
    return jnp.maximum(h, 0.0) + p * jnp.minimum(h, 0.0)


def _comb_dense_body(ah_ref, asc_ref, hwst_ref, als_ref, ald_ref, b_ref,
                     p_ref, w_ref, as_ref, ad_ref,
                     nhwst_ref, nals_ref, nald_ref):
    h = _h_block(ah_ref, asc_ref, hwst_ref, als_ref, ald_ref, b_ref, p_ref)
    hw = jnp.dot(h, w_ref[...], preferred_element_type=jnp.float32)
    _mm_out(hw, as_ref, ad_ref, nhwst_ref, nals_ref, nald_ref)


def _prev_specs(blk):
    return [
        pl.BlockSpec((NC, blk, DH), lambda i: (0, i, 0)),
        pl.BlockSpec((blk, SW), lambda i: (i, 0)),
        pl.BlockSpec((NC, blk, DH), lambda i: (0, i, 0)),
        pl.BlockSpec((blk, 1), lambda i: (i, 0)),
        pl.BlockSpec((blk, 1), lambda i: (i, 0)),
        pl.BlockSpec((1, D), lambda i: (0, 0)),
        pl.BlockSpec((1, 1), lambda i: (0, 0), memory_space=pltpu.SMEM),
    ]


def _comb_dense(acc_h, acc_s, hwst, als, ald, b, p, W, a_s, a_d, blk=2048):
    grid = (NPAD // blk,)
    return pl.pallas_call(
        _comb_dense_body,
        grid=grid,
        in_specs=_prev_specs(blk) + [
            pl.BlockSpec((D, D), lambda i: (0, 0)),
            pl.BlockSpec((D, 1), lambda i: (0, 0)),
            pl.BlockSpec((D, 1), lambda i: (0, 0)),
        ],
        out_specs=[
            pl.BlockSpec((NC, blk, DH), lambda i: (0, i, 0)),
            pl.BlockSpec((blk, 1), lambda i: (i, 0)),
            pl.BlockSpec((blk, 1), lambda i: (i, 0)),
        ],
        out_shape=[
            jax.ShapeDtypeStruct((NC, NPAD, DH), jnp.float32),
            jax.ShapeDtypeStruct((NPAD, 1), jnp.float32),
            jax.ShapeDtypeStruct((NPAD, 1), jnp.float32),
        ],
    )(acc_h, acc_s, hwst, als, ald, b.reshape(1, D), p.reshape(1, 1),
      W, a_s.reshape(D, 1), a_d.reshape(D, 1))


def _comb_final_body(ah_ref, asc_ref, hwst_ref, als_ref, ald_ref, b_ref,
                     p_ref, w_ref, bo_ref, out_ref):
    h = _h_block(ah_ref, asc_ref, hwst_ref, als_ref, ald_ref, b_ref, p_ref)
    out_ref[...] = jnp.dot(h, w_ref[...],
                           preferred_element_type=jnp.float32) + bo_ref[...]


def _comb_final(acc_h, acc_s, hwst, als, ald, b, p, Wout, bout, blk=2048):
    grid = (NPAD // blk,)
    return pl.pallas_call(
        _comb_final_body,
        grid=grid,
        in_specs=_prev_specs(blk) + [
            pl.BlockSpec((D, D), lambda i: (0, 0)),
            pl.BlockSpec((1, D), lambda i: (0, 0)),
        ],
        out_specs=pl.BlockSpec((blk, D), lambda i: (i, 0)),
        out_shape=jax.ShapeDtypeStruct((NPAD, D), jnp.float32),
    )(acc_h, acc_s, hwst, als, ald, b.reshape(1, D), p.reshape(1, 1),
      Wout, bout.reshape(1, D))


HALF = CPT // 2


def _sc_edge_body(hw_hbm, as_hbm, ad_hbm, src_hbm, dst_hbm,
                  out_h, out_s,
                  srcv, dstv, asv, adv, rows_a, rows_b, wbuf_a, wbuf_b, wvec,
                  acc_h, acc_s,
                  gsem_a, gsem_b, ssem_a, ssem_b, wsem_a, wsem_b):
    c = lax.axis_index("c")
    s = lax.axis_index("s")
    hw_c = hw_hbm.at[c]

    pltpu.sync_copy(as_hbm, asv)
    pltpu.sync_copy(ad_hbm, adv)

    def _zrow(r, _):
        zero16 = jnp.zeros((16,), jnp.float32)
        for v in range(DH // 16):
            rows_a[r, pl.ds(v * 16, 16)] = zero16
        wbuf_a[r, pl.ds(0, 16)] = zero16
        wbuf_b[r, pl.ds(0, 16)] = zero16
        return 0

    lax.fori_loop(0, CHUNK, _zrow, 0)

    for k in range(RPT // CHUNK):
        base = s * RPT + k * CHUNK
        pltpu.sync_copy(rows_a, acc_h.at[pl.ds(base, CHUNK)])
        pltpu.sync_copy(wbuf_a, acc_s.at[pl.ds(base, CHUNK)])
    plsc.subcore_barrier()

    def _score(ci, wb):
        @plsc.parallel_loop(0, CHUNK // 16, unroll=2)
        def _g(g):
            s16 = srcv[ci, pl.ds(g * 16, 16)]
            d16 = dstv[ci, pl.ds(g * 16, 16)]
            e = plsc.load_gather(asv, [s16]) + plsc.load_gather(adv, [d16])
            e = jnp.maximum(e, 0.0) + 0.2 * jnp.minimum(e, 0.0)
            w = jnp.exp(e)
            wvec[pl.ds(g * 16, 16)] = w
            r16 = g * 16 + lax.iota(jnp.int32, 16)
            plsc.store_scatter(wb, [r16, jnp.zeros((16,), jnp.int32)], w)

    def _scale(rows):
        @plsc.parallel_loop(0, CHUNK, unroll=16)
        def _r(r):
            wr = plsc.load_gather(wvec, [jnp.full((16,), r, jnp.int32)])
            for v in range(DH // 16):
                rows[r, pl.ds(v * 16, 16)] = rows[r, pl.ds(v * 16, 16)] * wr

    def _proc(ci, rows, wb, gsem, ssem, wsem, nrows, ngsem, nssem):
        @pl.when(ci + 1 < HALF)
        def _():
            @pl.when(ci >= 1)
            def _():
                pltpu.make_async_copy(nrows, acc_h.at[dstv.at[0]],
                                      nssem).wait()

            pltpu.async_copy(hw_c.at[srcv.at[ci + 1]], nrows, ngsem)

        @pl.when(jnp.logical_and(c == 0, ci >= 2))
        def _():
            pltpu.make_async_copy(wb, acc_s.at[dstv.at[0]], wsem).wait()

        pltpu.make_async_copy(hw_c.at[srcv.at[0]], rows, gsem).wait()
        _score(ci, wb)
        _scale(rows)

        didx = dstv.at[ci]

        @pl.when(c == 0)
        def _():
            pltpu.async_copy(wb, acc_s.at[didx], wsem, add=True)

        pltpu.async_copy(rows, acc_h.at[didx], ssem, add=True)

    for h0 in (0, HALF):
        pltpu.sync_copy(src_hbm.at[s, pl.ds(h0, HALF)], srcv)
        pltpu.sync_copy(dst_hbm.at[s, pl.ds(h0, HALF)], dstv)

        pltpu.async_copy(hw_c.at[srcv.at[0]], rows_a, gsem_a)

        def _body(ci, _):
            @pl.when(ci % 2 == 0)
            def _():
                _proc(ci, rows_a, wbuf_a, gsem_a, ssem_a, wsem_a,
                      rows_b, gsem_b, ssem_b)

            @pl.when(ci % 2 == 1)
            def _():
                _proc(ci, rows_b, wbuf_b, gsem_b, ssem_b, wsem_b,
                      rows_a, gsem_a, ssem_a)

            return 0

        lax.fori_loop(0, HALF, _body, 0)

        pltpu.make_async_copy(rows_a, acc_h.at[dstv.at[0]], ssem_a).wait()
        pltpu.make_async_copy(rows_b, acc_h.at[dstv.at[0]], ssem_b).wait()

        @pl.when(c == 0)
        def _():
            pltpu.make_async_copy(wbuf_a, acc_s.at[dstv.at[0]], wsem_a).wait()
            pltpu.make_async_copy(wbuf_b, acc_s.at[dstv.at[0]], wsem_b).wait()

    plsc.subcore_barrier()

    base = s * RPT
    pltpu.sync_copy(acc_h.at[pl.ds(base, RPT)], out_h.at[c].at[pl.ds(base, RPT)])

    @pl.when(c == 0)
    def _():
        pltpu.sync_copy(acc_s.at[pl.ds(base, RPT)], out_s.at[pl.ds(base, RPT)])


@functools.partial(
    pl.kernel,
    out_type=(
        jax.ShapeDtypeStruct((NC, NPAD, DH), jnp.float32),
        jax.ShapeDtypeStruct((NPAD, SW), jnp.float32),
    ),
    mesh=plsc.VectorSubcoreMesh(core_axis_name="c", subcore_axis_name="s"),
    compiler_params=pltpu.CompilerParams(needs_layout_passes=False,
                                         use_tc_tiling_on_sc=False),
    scratch_types=[
        pltpu.VMEM((HALF, CHUNK), jnp.int32),
        pltpu.VMEM((HALF, CHUNK), jnp.int32),
        pltpu.VMEM((NPAD,), jnp.float32),
        pltpu.VMEM((NPAD,), jnp.float32),
        pltpu.VMEM((CHUNK, DH), jnp.float32),
        pltpu.VMEM((CHUNK, DH), jnp.float32),
        pltpu.VMEM((CHUNK, SW), jnp.float32),
        pltpu.VMEM((CHUNK, SW), jnp.float32),
        pltpu.VMEM((CHUNK,), jnp.float32),
        pltpu.VMEM_SHARED((NPAD, DH), jnp.float32),
        pltpu.VMEM_SHARED((NPAD, SW), jnp.float32),
        pltpu.SemaphoreType.DMA,
        pltpu.SemaphoreType.DMA,
        pltpu.SemaphoreType.DMA,
        pltpu.SemaphoreType.DMA,
        pltpu.SemaphoreType.DMA,
        pltpu.SemaphoreType.DMA,
    ],
)
def _sc_edge(hw_hbm, as_hbm, ad_hbm, src_hbm, dst_hbm, out_h, out_s,
             srcv, dstv, asv, adv, rows_a, rows_b, wbuf_a, wbuf_b, wvec,
             acc_h, acc_s, gsem_a, gsem_b, ssem_a, ssem_b,
             wsem_a, wsem_b):
    _sc_edge_body(hw_hbm, as_hbm, ad_hbm, src_hbm, dst_hbm, out_h, out_s,
                  srcv, dstv, asv, adv, rows_a, rows_b, wbuf_a, wbuf_b, wvec,
                  acc_h, acc_s,
                  gsem_a, gsem_b, ssem_a, ssem_b, wsem_a, wsem_b)


def kernel(x, edge_index, edge_weight, emb, Wout, bout,
           W1, a_src1, a_dst1, b1, p1,
           W2, a_src2, a_dst2, b2, p2,
           W3, a_src3, a_dst3, b3, p3):
    h = jnp.pad(emb[x], ((0, NPAD - N), (0, 0)))
    srcp = jnp.pad(edge_index[0], (0, EPAD - E)).reshape(NS, CPT, CHUNK)
    dstp = jnp.pad(edge_index[1], (0, EPAD - E),
                   constant_values=TRASH).reshape(NS, CPT, CHUNK)

    hwst, als, ald = _dense(h, W1, a_src1, a_dst1)
    acc_h, acc_s = _sc_edge(hwst, als.reshape(NPAD), ald.reshape(NPAD),
                            srcp, dstp)

    for b, p, W, a_s, a_d in ((b1, p1, W2, a_src2, a_dst2),
                              (b2, p2, W3, a_src3, a_dst3)):
        nhwst, nals, nald = _comb_dense(acc_h, acc_s, hwst, als, ald,
                                        b, p, W, a_s, a_d)
        hwst, als, ald = nhwst, nals, nald
        acc_h, acc_s = _sc_edge(hwst, als.reshape(NPAD), ald.reshape(NPAD),
                                srcp, dstp)

    out = _comb_final(acc_h, acc_s, hwst, als, ald, b3, p3, Wout, bout)
    return out[:N]


# speedup vs baseline: 1.0001x; 1.0001x over previous
import functools

import jax
import jax.numpy as jnp
from jax import lax
from jax.experimental import pallas as pl
from jax.experimental.pallas import tpu as pltpu
from jax.experimental.pallas import tpu_sc as plsc

N = 10000
D = 128
E = 320000
NPAD = 10240
TRASH = 10000
NC = 2
NS = 16
DH = D // NC
CHUNK = 128
CPT = 158
EPT = CPT * CHUNK
EPAD = NS * EPT
RPT = NPAD // NS
SW = 16


def _mm_out(hw, as_ref, ad_ref, hwst_ref, als_ref, ald_ref):
    hwst_ref[0] = hw[:, :DH]
    hwst_ref[1] = hw[:, DH:]
    als_ref[...] = jnp.dot(hw, as_ref[...], preferred_element_type=jnp.float32)
    ald_ref[...] = jnp.dot(hw, ad_ref[...], preferred_element_type=jnp.float32)


def _dense_body(h_ref, w_ref, as_ref, ad_ref, hwst_ref, als_ref, ald_ref):
    hw = jnp.dot(h_ref[...], w_ref[...], preferred_element_type=jnp.float32)
    _mm_out(hw, as_ref, ad_ref, hwst_ref, als_ref, ald_ref)


def _dense(h, W, a_s, a_d, blk=2048):
    grid = (NPAD // blk,)
    return pl.pallas_call(
        _dense_body,
        grid=grid,
        in_specs=[
            pl.BlockSpec((blk, D), lambda i: (i, 0)),
            pl.BlockSpec((D, D), lambda i: (0, 0)),
            pl.BlockSpec((D, 1), lambda i: (0, 0)),
            pl.BlockSpec((D, 1), lambda i: (0, 0)),
        ],
        out_specs=[
            pl.BlockSpec((NC, blk, DH), lambda i: (0, i, 0)),
            pl.BlockSpec((blk, 1), lambda i: (i, 0)),
            pl.BlockSpec((blk, 1), lambda i: (i, 0)),
        ],
        out_shape=[
            jax.ShapeDtypeStruct((NC, NPAD, DH), jnp.float32),
            jax.ShapeDtypeStruct((NPAD, 1), jnp.float32),
            jax.ShapeDtypeStruct((NPAD, 1), jnp.float32),
        ],
    )(h, W, a_s.reshape(D, 1), a_d.reshape(D, 1))


def _h_block(ah_ref, asc_ref, hwst_ref, als_ref, ald_ref, b_ref, p_ref):
    e = als_ref[...] + ald_ref[...]
    wself = jnp.exp(jnp.maximum(e, 0.0) + 0.2 * jnp.minimum(e, 0.0))
    hw = jnp.concatenate([hwst_ref[0], hwst_ref[1]], axis=-1)
    num = ah_ref[...] + wself * hw
    den = asc_ref[:, 0:1] + wself
    h = num / (den + 1e-16) + b_ref[...]
    p = p_ref[0, 0]
    return jnp.maximum(h, 0.0) + p * jnp.minimum(h, 0.0)


def _comb_dense_body(ah_ref, asc_ref, hwst_ref, als_ref, ald_ref, b_ref,
                     p_ref, w_ref, as_ref, ad_ref,
                     nhwst_ref, nals_ref, nald_ref):
    h = _h_block(ah_ref, asc_ref, hwst_ref, als_ref, ald_ref, b_ref, p_ref)
    hw = jnp.dot(h, w_ref[...], preferred_element_type=jnp.float32)
    _mm_out(hw, as_ref, ad_ref, nhwst_ref, nals_ref, nald_ref)


def _prev_specs(blk):
    return [
        pl.BlockSpec((blk, D), lambda i: (i, 0)),
        pl.BlockSpec((blk, SW), lambda i: (i, 0)),
        pl.BlockSpec((NC, blk, DH), lambda i: (0, i, 0)),
        pl.BlockSpec((blk, 1), lambda i: (i, 0)),
        pl.BlockSpec((blk, 1), lambda i: (i, 0)),
        pl.BlockSpec((1, D), lambda i: (0, 0)),
        pl.BlockSpec((1, 1), lambda i: (0, 0), memory_space=pltpu.SMEM),
    ]


def _comb_dense(acc_h, acc_s, hwst, als, ald, b, p, W, a_s, a_d, blk=2048):
    grid = (NPAD // blk,)
    return pl.pallas_call(
        _comb_dense_body,
        grid=grid,
        in_specs=_prev_specs(blk) + [
            pl.BlockSpec((D, D), lambda i: (0, 0)),
            pl.BlockSpec((D, 1), lambda i: (0, 0)),
            pl.BlockSpec((D, 1), lambda i: (0, 0)),
        ],
        out_specs=[
            pl.BlockSpec((NC, blk, DH), lambda i: (0, i, 0)),
            pl.BlockSpec((blk, 1), lambda i: (i, 0)),
            pl.BlockSpec((blk, 1), lambda i: (i, 0)),
        ],
        out_shape=[
            jax.ShapeDtypeStruct((NC, NPAD, DH), jnp.float32),
            jax.ShapeDtypeStruct((NPAD, 1), jnp.float32),
            jax.ShapeDtypeStruct((NPAD, 1), jnp.float32),
        ],
    )(acc_h, acc_s, hwst, als, ald, b.reshape(1, D), p.reshape(1, 1),
      W, a_s.reshape(D, 1), a_d.reshape(D, 1))


def _comb_final_body(ah_ref, asc_ref, hwst_ref, als_ref, ald_ref, b_ref,
                     p_ref, w_ref, bo_ref, out_ref):
    h = _h_block(ah_ref, asc_ref, hwst_ref, als_ref, ald_ref, b_ref, p_ref)
    out_ref[...] = jnp.dot(h, w_ref[...],
                           preferred_element_type=jnp.float32) + bo_ref[...]


def _comb_final(acc_h, acc_s, hwst, als, ald, b, p, Wout, bout, blk=2048):
    grid = (NPAD // blk,)
    return pl.pallas_call(
        _comb_final_body,
        grid=grid,
        in_specs=_prev_specs(blk) + [
            pl.BlockSpec((D, D), lambda i: (0, 0)),
            pl.BlockSpec((1, D), lambda i: (0, 0)),
        ],
        out_specs=pl.BlockSpec((blk, D), lambda i: (i, 0)),
        out_shape=jax.ShapeDtypeStruct((NPAD, D), jnp.float32),
    )(acc_h, acc_s, hwst, als, ald, b.reshape(1, D), p.reshape(1, 1),
      Wout, bout.reshape(1, D))


HALF = CPT // 2


def _sc_edge_body(hw_hbm, as_hbm, ad_hbm, src_hbm, dst_hbm,
                  out_h, out_s,
                  srcv, dstv, asv, adv, rows_a, rows_b, wbuf_a, wbuf_b, wvec,
                  acc_h, acc_s,
                  gsem_a, gsem_b, ssem_a, ssem_b, wsem_a, wsem_b):
    c = lax.axis_index("c")
    s = lax.axis_index("s")
    hw_c = hw_hbm.at[c]

    pltpu.sync_copy(as_hbm, asv)
    pltpu.sync_copy(ad_hbm, adv)

    def _zrow(r, _):
        zero16 = jnp.zeros((16,), jnp.float32)
        for v in range(DH // 16):
            rows_a[r, pl.ds(v * 16, 16)] = zero16
        wbuf_a[r, pl.ds(0, 16)] = zero16
        wbuf_b[r, pl.ds(0, 16)] = zero16
        return 0

    lax.fori_loop(0, CHUNK, _zrow, 0)

    for k in range(RPT // CHUNK):
        base = s * RPT + k * CHUNK
        pltpu.sync_copy(rows_a, acc_h.at[pl.ds(base, CHUNK)])
        pltpu.sync_copy(wbuf_a, acc_s.at[pl.ds(base, CHUNK)])
    plsc.subcore_barrier()

    def _score(ci, wb):
        @plsc.parallel_loop(0, CHUNK // 16, unroll=2)
        def _g(g):
            s16 = srcv[ci, pl.ds(g * 16, 16)]
            d16 = dstv[ci, pl.ds(g * 16, 16)]
            e = plsc.load_gather(asv, [s16]) + plsc.load_gather(adv, [d16])
            e = jnp.maximum(e, 0.0) + 0.2 * jnp.minimum(e, 0.0)
            w = jnp.exp(e)
            wvec[pl.ds(g * 16, 16)] = w
            r16 = g * 16 + lax.iota(jnp.int32, 16)
            plsc.store_scatter(wb, [r16, jnp.zeros((16,), jnp.int32)], w)

    def _scale(rows):
        @plsc.parallel_loop(0, CHUNK, unroll=8)
        def _r(r):
            wr = plsc.load_gather(wvec, [jnp.full((16,), r, jnp.int32)])
            for v in range(DH // 16):
                rows[r, pl.ds(v * 16, 16)] = rows[r, pl.ds(v * 16, 16)] * wr

    def _proc(ci, rows, wb, gsem, ssem, wsem, nrows, ngsem, nssem):
        @pl.when(ci + 1 < HALF)
        def _():
            @pl.when(ci >= 1)
            def _():
                pltpu.make_async_copy(nrows, acc_h.at[dstv.at[0]],
                                      nssem).wait()

            pltpu.async_copy(hw_c.at[srcv.at[ci + 1]], nrows, ngsem)

        @pl.when(jnp.logical_and(c == 0, ci >= 2))
        def _():
            pltpu.make_async_copy(wb, acc_s.at[dstv.at[0]], wsem).wait()

        pltpu.make_async_copy(hw_c.at[srcv.at[0]], rows, gsem).wait()
        _score(ci, wb)
        _scale(rows)

        didx = dstv.at[ci]

        @pl.when(c == 0)
        def _():
            pltpu.async_copy(wb, acc_s.at[didx], wsem, add=True)

        pltpu.async_copy(rows, acc_h.at[didx], ssem, add=True)

    for h0 in (0, HALF):
        pltpu.sync_copy(src_hbm.at[s, pl.ds(h0, HALF)], srcv)
        pltpu.sync_copy(dst_hbm.at[s, pl.ds(h0, HALF)], dstv)

        pltpu.async_copy(hw_c.at[srcv.at[0]], rows_a, gsem_a)

        def _body(ci, _):
            @pl.when(ci % 2 == 0)
            def _():
                _proc(ci, rows_a, wbuf_a, gsem_a, ssem_a, wsem_a,
                      rows_b, gsem_b, ssem_b)

            @pl.when(ci % 2 == 1)
            def _():
                _proc(ci, rows_b, wbuf_b, gsem_b, ssem_b, wsem_b,
                      rows_a, gsem_a, ssem_a)

            return 0

        lax.fori_loop(0, HALF, _body, 0)

        pltpu.make_async_copy(rows_a, acc_h.at[dstv.at[0]], ssem_a).wait()
        pltpu.make_async_copy(rows_b, acc_h.at[dstv.at[0]], ssem_b).wait()

        @pl.when(c == 0)
        def _():
            pltpu.make_async_copy(wbuf_a, acc_s.at[dstv.at[0]], wsem_a).wait()
            pltpu.make_async_copy(wbuf_b, acc_s.at[dstv.at[0]], wsem_b).wait()

    plsc.subcore_barrier()

    base = s * RPT
    pltpu.sync_copy(acc_h.at[pl.ds(base, RPT)],
                    out_h.at[pl.ds(base, RPT), pl.ds(c * DH, DH)])

    @pl.when(c == 0)
    def _():
        pltpu.sync_copy(acc_s.at[pl.ds(base, RPT)], out_s.at[pl.ds(base, RPT)])


@functools.partial(
    pl.kernel,
    out_type=(
        jax.ShapeDtypeStruct((NPAD, D), jnp.float32),
        jax.ShapeDtypeStruct((NPAD, SW), jnp.float32),
    ),
    mesh=plsc.VectorSubcoreMesh(core_axis_name="c", subcore_axis_name="s"),
    compiler_params=pltpu.CompilerParams(needs_layout_passes=False,
                                         use_tc_tiling_on_sc=False),
    scratch_types=[
        pltpu.VMEM((HALF, CHUNK), jnp.int32),
        pltpu.VMEM((HALF, CHUNK), jnp.int32),
        pltpu.VMEM((NPAD,), jnp.float32),
        pltpu.VMEM((NPAD,), jnp.float32),
        pltpu.VMEM((CHUNK, DH), jnp.float32),
        pltpu.VMEM((CHUNK, DH), jnp.float32),
        pltpu.VMEM((CHUNK, SW), jnp.float32),
        pltpu.VMEM((CHUNK, SW), jnp.float32),
        pltpu.VMEM((CHUNK,), jnp.float32),
        pltpu.VMEM_SHARED((NPAD, DH), jnp.float32),
        pltpu.VMEM_SHARED((NPAD, SW), jnp.float32),
        pltpu.SemaphoreType.DMA,
        pltpu.SemaphoreType.DMA,
        pltpu.SemaphoreType.DMA,
        pltpu.SemaphoreType.DMA,
        pltpu.SemaphoreType.DMA,
        pltpu.SemaphoreType.DMA,
    ],
)
def _sc_edge(hw_hbm, as_hbm, ad_hbm, src_hbm, dst_hbm, out_h, out_s,
             srcv, dstv, asv, adv, rows_a, rows_b, wbuf_a, wbuf_b, wvec,
             acc_h, acc_s, gsem_a, gsem_b, ssem_a, ssem_b,
             wsem_a, wsem_b):
    _sc_edge_body(hw_hbm, as_hbm, ad_hbm, src_hbm, dst_hbm, out_h, out_s,
                  srcv, dstv, asv, adv, rows_a, rows_b, wbuf_a, wbuf_b, wvec,
                  acc_h, acc_s,
                  gsem_a, gsem_b, ssem_a, ssem_b, wsem_a, wsem_b)


def kernel(x, edge_index, edge_weight, emb, Wout, bout,
           W1, a_src1, a_dst1, b1, p1,
           W2, a_src2, a_dst2, b2, p2,
           W3, a_src3, a_dst3, b3, p3):
    h = jnp.pad(emb[x], ((0, NPAD - N), (0, 0)))
    srcp = jnp.pad(edge_index[0], (0, EPAD - E)).reshape(NS, CPT, CHUNK)
    dstp = jnp.pad(edge_index[1], (0, EPAD - E),
                   constant_values=TRASH).reshape(NS, CPT, CHUNK)

    hwst, als, ald = _dense(h, W1, a_src1, a_dst1)
    acc_h, acc_s = _sc_edge(hwst, als.reshape(NPAD), ald.reshape(NPAD),
                            srcp, dstp)

    for b, p, W, a_s, a_d in ((b1, p1, W2, a_src2, a_dst2),
                              (b2, p2, W3, a_src3, a_dst3)):
        nhwst, nals, nald = _comb_dense(acc_h, acc_s, hwst, als, ald,
                                        b, p, W, a_s, a_d)
        hwst, als, ald = nhwst, nals, nald
        acc_h, acc_s = _sc_edge(hwst, als.reshape(NPAD), ald.reshape(NPAD),
                                srcp, dstp)

    out = _comb_final(acc_h, acc_s, hwst, als, ald, b3, p3, Wout, bout)
    return out[:N]


# speedup vs baseline: 1.0228x; 1.0226x over previous
import functools

import jax
import jax.numpy as jnp
from jax import lax
from jax.experimental import pallas as pl
from jax.experimental.pallas import tpu as pltpu
from jax.experimental.pallas import tpu_sc as plsc

N = 10000
D = 128
E = 320000
NPAD = 10240
TRASH = 10000
NC = 2
NS = 16
DH = D // NC
CHUNK = 128
CPT = 158
EPT = CPT * CHUNK
EPAD = NS * EPT
RPT = NPAD // NS
SW = 16


def _mm_out(hw, as_ref, ad_ref, hwst_ref, als_ref, ald_ref):
    hwst_ref[0] = hw[:, :DH]
    hwst_ref[1] = hw[:, DH:]
    als_ref[...] = jnp.dot(hw, as_ref[...], preferred_element_type=jnp.float32)
    ald_ref[...] = jnp.dot(hw, ad_ref[...], preferred_element_type=jnp.float32)


def _dense_body(h_ref, w_ref, as_ref, ad_ref, hwst_ref, als_ref, ald_ref):
    hw = jnp.dot(h_ref[...], w_ref[...], preferred_element_type=jnp.float32)
    _mm_out(hw, as_ref, ad_ref, hwst_ref, als_ref, ald_ref)


def _dense(h, W, a_s, a_d, blk=2048):
    grid = (NPAD // blk,)
    return pl.pallas_call(
        _dense_body,
        grid=grid,
        in_specs=[
            pl.BlockSpec((blk, D), lambda i: (i, 0)),
            pl.BlockSpec((D, D), lambda i: (0, 0)),
            pl.BlockSpec((D, 1), lambda i: (0, 0)),
            pl.BlockSpec((D, 1), lambda i: (0, 0)),
        ],
        out_specs=[
            pl.BlockSpec((NC, blk, DH), lambda i: (0, i, 0)),
            pl.BlockSpec((blk, 1), lambda i: (i, 0)),
            pl.BlockSpec((blk, 1), lambda i: (i, 0)),
        ],
        out_shape=[
            jax.ShapeDtypeStruct((NC, NPAD, DH), jnp.float32),
            jax.ShapeDtypeStruct((NPAD, 1), jnp.float32),
            jax.ShapeDtypeStruct((NPAD, 1), jnp.float32),
        ],
    )(h, W, a_s.reshape(D, 1), a_d.reshape(D, 1))


def _h_block(ah_ref, asc_ref, hwst_ref, als_ref, ald_ref, b_ref, p_ref):
    e = als_ref[...] + ald_ref[...]
    wself = jnp.exp(jnp.maximum(e, 0.0) + 0.2 * jnp.minimum(e, 0.0))
    hw = jnp.concatenate([hwst_ref[0], hwst_ref[1]], axis=-1)
    num = ah_ref[...] + wself * hw
    den = asc_ref[0, :, 0:1] + asc_ref[1, :, 0:1] + wself
    h = num / (den + 1e-16) + b_ref[...]
    p = p_ref[0, 0]
    return jnp.maximum(h, 0.0) + p * jnp.minimum(h, 0.0)


def _comb_dense_body(ah_ref, asc_ref, hwst_ref, als_ref, ald_ref, b_ref,
                     p_ref, w_ref, as_ref, ad_ref,
                     nhwst_ref, nals_ref, nald_ref):
    h = _h_block(ah_ref, asc_ref, hwst_ref, als_ref, ald_ref, b_ref, p_ref)
    hw = jnp.dot(h, w_ref[...], preferred_element_type=jnp.float32)
    _mm_out(hw, as_ref, ad_ref, nhwst_ref, nals_ref, nald_ref)


def _prev_specs(blk):
    return [
        pl.BlockSpec((blk, D), lambda i: (i, 0)),
        pl.BlockSpec((NC, blk, SW), lambda i: (0, i, 0)),
        pl.BlockSpec((NC, blk, DH), lambda i: (0, i, 0)),
        pl.BlockSpec((blk, 1), lambda i: (i, 0)),
        pl.BlockSpec((blk, 1), lambda i: (i, 0)),
        pl.BlockSpec((1, D), lambda i: (0, 0)),
        pl.BlockSpec((1, 1), lambda i: (0, 0), memory_space=pltpu.SMEM),
    ]


def _comb_dense(acc_h, acc_s, hwst, als, ald, b, p, W, a_s, a_d, blk=2048):
    grid = (NPAD // blk,)
    return pl.pallas_call(
        _comb_dense_body,
        grid=grid,
        in_specs=_prev_specs(blk) + [
            pl.BlockSpec((D, D), lambda i: (0, 0)),
            pl.BlockSpec((D, 1), lambda i: (0, 0)),
            pl.BlockSpec((D, 1), lambda i: (0, 0)),
        ],
        out_specs=[
            pl.BlockSpec((NC, blk, DH), lambda i: (0, i, 0)),
            pl.BlockSpec((blk, 1), lambda i: (i, 0)),
            pl.BlockSpec((blk, 1), lambda i: (i, 0)),
        ],
        out_shape=[
            jax.ShapeDtypeStruct((NC, NPAD, DH), jnp.float32),
            jax.ShapeDtypeStruct((NPAD, 1), jnp.float32),
            jax.ShapeDtypeStruct((NPAD, 1), jnp.float32),
        ],
    )(acc_h, acc_s, hwst, als, ald, b.reshape(1, D), p.reshape(1, 1),
      W, a_s.reshape(D, 1), a_d.reshape(D, 1))


def _comb_final_body(ah_ref, asc_ref, hwst_ref, als_ref, ald_ref, b_ref,
                     p_ref, w_ref, bo_ref, out_ref):
    h = _h_block(ah_ref, asc_ref, hwst_ref, als_ref, ald_ref, b_ref, p_ref)
    out_ref[...] = jnp.dot(h, w_ref[...],
                           preferred_element_type=jnp.float32) + bo_ref[...]


def _comb_final(acc_h, acc_s, hwst, als, ald, b, p, Wout, bout, blk=2048):
    grid = (NPAD // blk,)
    return pl.pallas_call(
        _comb_final_body,
        grid=grid,
        in_specs=_prev_specs(blk) + [
            pl.BlockSpec((D, D), lambda i: (0, 0)),
            pl.BlockSpec((1, D), lambda i: (0, 0)),
        ],
        out_specs=pl.BlockSpec((blk, D), lambda i: (i, 0)),
        out_shape=jax.ShapeDtypeStruct((NPAD, D), jnp.float32),
    )(acc_h, acc_s, hwst, als, ald, b.reshape(1, D), p.reshape(1, 1),
      Wout, bout.reshape(1, D))


HALF = CPT // 2


def _sc_edge_body(hw_hbm, as_hbm, ad_hbm, src_hbm, dst_hbm,
                  out_h, out_s,
                  srcv, dstv, asv, adv, rows_a, rows_b, wbuf_a, wbuf_b, wvec,
                  acc_h, acc_s,
                  gsem_a, gsem_b, ssem_a, ssem_b, wsem_a, wsem_b):
    c = lax.axis_index("c")
    s = lax.axis_index("s")
    hw_c = hw_hbm.at[c]

    pltpu.sync_copy(as_hbm, asv)
    pltpu.sync_copy(ad_hbm, adv)

    def _zrow(r, _):
        zero16 = jnp.zeros((16,), jnp.float32)
        for v in range(DH // 16):
            rows_a[r, pl.ds(v * 16, 16)] = zero16
        wbuf_a[r, pl.ds(0, 16)] = zero16
        wbuf_b[r, pl.ds(0, 16)] = zero16
        return 0

    lax.fori_loop(0, CHUNK, _zrow, 0)

    for k in range(RPT // CHUNK):
        base = s * RPT + k * CHUNK
        pltpu.sync_copy(rows_a, acc_h.at[pl.ds(base, CHUNK)])
        pltpu.sync_copy(wbuf_a, acc_s.at[pl.ds(base, CHUNK)])
    plsc.subcore_barrier()

    def _score(ci, wb):
        @plsc.parallel_loop(0, CHUNK // 16, unroll=2)
        def _g(g):
            s16 = srcv[ci, pl.ds(g * 16, 16)]
            d16 = dstv[ci, pl.ds(g * 16, 16)]
            e = plsc.load_gather(asv, [s16]) + plsc.load_gather(adv, [d16])
            e = jnp.maximum(e, 0.0) + 0.2 * jnp.minimum(e, 0.0)
            w = jnp.exp(e)
            wvec[pl.ds(g * 16, 16)] = w
            r16 = g * 16 + lax.iota(jnp.int32, 16)
            plsc.store_scatter(wb, [r16, jnp.zeros((16,), jnp.int32)], w)

    def _scale(rows):
        @plsc.parallel_loop(0, CHUNK, unroll=8)
        def _r(r):
            wr = plsc.load_gather(wvec, [jnp.full((16,), r, jnp.int32)])
            for v in range(DH // 16):
                rows[r, pl.ds(v * 16, 16)] = rows[r, pl.ds(v * 16, 16)] * wr

    def _proc(ci, rows, wb, gsem, ssem, wsem, nrows, ngsem, nssem):
        @pl.when(ci + 1 < HALF)
        def _():
            @pl.when(ci >= 1)
            def _():
                pltpu.make_async_copy(nrows, acc_h.at[dstv.at[0]],
                                      nssem).wait()

            pltpu.async_copy(hw_c.at[srcv.at[ci + 1]], nrows, ngsem)

        @pl.when(jnp.logical_and(ci % 2 == c, ci >= 2))
        def _():
            pltpu.make_async_copy(wb, acc_s.at[dstv.at[0]], wsem).wait()

        pltpu.make_async_copy(hw_c.at[srcv.at[0]], rows, gsem).wait()
        _score(ci, wb)
        _scale(rows)

        didx = dstv.at[ci]

        @pl.when(ci % 2 == c)
        def _():
            pltpu.async_copy(wb, acc_s.at[didx], wsem, add=True)

        pltpu.async_copy(rows, acc_h.at[didx], ssem, add=True)

    for h0 in (0, HALF):
        pltpu.sync_copy(src_hbm.at[s, pl.ds(h0, HALF)], srcv)
        pltpu.sync_copy(dst_hbm.at[s, pl.ds(h0, HALF)], dstv)

        pltpu.async_copy(hw_c.at[srcv.at[0]], rows_a, gsem_a)

        def _body(ci, _):
            @pl.when(ci % 2 == 0)
            def _():
                _proc(ci, rows_a, wbuf_a, gsem_a, ssem_a, wsem_a,
                      rows_b, gsem_b, ssem_b)

            @pl.when(ci % 2 == 1)
            def _():
                _proc(ci, rows_b, wbuf_b, gsem_b, ssem_b, wsem_b,
                      rows_a, gsem_a, ssem_a)

            return 0

        lax.fori_loop(0, HALF, _body, 0)

        pltpu.make_async_copy(rows_a, acc_h.at[dstv.at[0]], ssem_a).wait()
        pltpu.make_async_copy(rows_b, acc_h.at[dstv.at[0]], ssem_b).wait()

        @pl.when(c == 0)
        def _():
            pltpu.make_async_copy(wbuf_a, acc_s.at[dstv.at[0]], wsem_a).wait()

        @pl.when(c == 1)
        def _():
            pltpu.make_async_copy(wbuf_b, acc_s.at[dstv.at[0]], wsem_b).wait()

    plsc.subcore_barrier()

    base = s * RPT
    pltpu.sync_copy(acc_h.at[pl.ds(base, RPT)],
                    out_h.at[pl.ds(base, RPT), pl.ds(c * DH, DH)])
    pltpu.sync_copy(acc_s.at[pl.ds(base, RPT)],
                    out_s.at[c, pl.ds(base, RPT)])


@functools.partial(
    pl.kernel,
    out_type=(
        jax.ShapeDtypeStruct((NPAD, D), jnp.float32),
        jax.ShapeDtypeStruct((NC, NPAD, SW), jnp.float32),
    ),
    mesh=plsc.VectorSubcoreMesh(core_axis_name="c", subcore_axis_name="s"),
    compiler_params=pltpu.CompilerParams(needs_layout_passes=False,
                                         use_tc_tiling_on_sc=False),
    scratch_types=[
        pltpu.VMEM((HALF, CHUNK), jnp.int32),
        pltpu.VMEM((HALF, CHUNK), jnp.int32),
        pltpu.VMEM((NPAD,), jnp.float32),
        pltpu.VMEM((NPAD,), jnp.float32),
        pltpu.VMEM((CHUNK, DH), jnp.float32),
        pltpu.VMEM((CHUNK, DH), jnp.float32),
        pltpu.VMEM((CHUNK, SW), jnp.float32),
        pltpu.VMEM((CHUNK, SW), jnp.float32),
        pltpu.VMEM((CHUNK,), jnp.float32),
        pltpu.VMEM_SHARED((NPAD, DH), jnp.float32),
        pltpu.VMEM_SHARED((NPAD, SW), jnp.float32),
        pltpu.SemaphoreType.DMA,
        pltpu.SemaphoreType.DMA,
        pltpu.SemaphoreType.DMA,
        pltpu.SemaphoreType.DMA,
        pltpu.SemaphoreType.DMA,
        pltpu.SemaphoreType.DMA,
    ],
)
def _sc_edge(hw_hbm, as_hbm, ad_hbm, src_hbm, dst_hbm, out_h, out_s,
             srcv, dstv, asv, adv, rows_a, rows_b, wbuf_a, wbuf_b, wvec,
             acc_h, acc_s, gsem_a, gsem_b, ssem_a, ssem_b,
             wsem_a, wsem_b):
    _sc_edge_body(hw_hbm, as_hbm, ad_hbm, src_hbm, dst_hbm, out_h, out_s,
                  srcv, dstv, asv, adv, rows_a, rows_b, wbuf_a, wbuf_b, wvec,
                  acc_h, acc_s,
                  gsem_a, gsem_b, ssem_a, ssem_b, wsem_a, wsem_b)


def kernel(x, edge_index, edge_weight, emb, Wout, bout,
           W1, a_src1, a_dst1, b1, p1,
           W2, a_src2, a_dst2, b2, p2,
           W3, a_src3, a_dst3, b3, p3):
    h = jnp.pad(emb[x], ((0, NPAD - N), (0, 0)))
    srcp = jnp.pad(edge_index[0], (0, EPAD - E)).reshape(NS, CPT, CHUNK)
    dstp = jnp.pad(edge_index[1], (0, EPAD - E),
                   constant_values=TRASH).reshape(NS, CPT, CHUNK)

    hwst, als, ald = _dense(h, W1, a_src1, a_dst1)
    acc_h, acc_s = _sc_edge(hwst, als.reshape(NPAD), ald.reshape(NPAD),
                            srcp, dstp)

    for b, p, W, a_s, a_d in ((b1, p1, W2, a_src2, a_dst2),
                              (b2, p2, W3, a_src3, a_dst3)):
        nhwst, nals, nald = _comb_dense(acc_h, acc_s, hwst, als, ald,
                                        b, p, W, a_s, a_d)
        hwst, als, ald = nhwst, nals, nald
        acc_h, acc_s = _sc_edge(hwst, als.reshape(NPAD), ald.reshape(NPAD),
                                srcp, dstp)

    out = _comb_final(acc_h, acc_s, hwst, als, ald, b3, p3, Wout, bout)
    return out[:N]
